# Initial kernel scaffold; baseline (speedup 1.0000x reference)
import jax
import jax.numpy as jnp
from jax.experimental import pallas as pl


def kernel(x, edge_index, W_hops, b_hops, ln_g_hops, ln_b_hops, W_out, b_out, ln_g_out, ln_b_out):
    raise NotImplementedError("write your pallas kernel here")


# speedup vs baseline: 8.2321x; 8.2321x over previous
import functools

import jax
import jax.numpy as jnp
from jax import lax
from jax.experimental import pallas as pl
from jax.experimental.pallas import tpu as pltpu, tpu_sc as plsc

NC = 2
NS = 16
CHUNK = 128


def _round_up(a, b):
    return (a + b - 1) // b * b


def _make_deg_kernel(n_pad, e_pad, d, interpret=False):
    edges_per_tile = e_pad // (NC * NS)
    n_chunks = edges_per_tile // CHUNK
    rows_per_tile = n_pad // NS
    mesh = plsc.VectorSubcoreMesh(core_axis_name="c", subcore_axis_name="s", num_cores=NC, num_subcores=NS)

    @functools.partial(
        pl.kernel,
        mesh=mesh,
        out_type=jax.ShapeDtypeStruct((NC, n_pad, d), jnp.float32),
        scratch_types=[
            pltpu.VMEM_SHARED((n_pad, d), jnp.float32),
            pltpu.VMEM((CHUNK,), jnp.int32),
            pltpu.VMEM((CHUNK, d), jnp.float32),
        ],
        interpret=interpret,
    )
    def deg_kernel(dst_hbm, ones_hbm, zeros_hbm, out_hbm, acc, dstv, ones_v):
        c = lax.axis_index("c")
        s = lax.axis_index("s")
        pltpu.sync_copy(zeros_hbm.at[pl.ds(s * rows_per_tile, rows_per_tile)],
                        acc.at[pl.ds(s * rows_per_tile, rows_per_tile)])
        pltpu.sync_copy(ones_hbm, ones_v)
        plsc.subcore_barrier()
        base = c * (e_pad // NC) + s * edges_per_tile

        def body(j, carry):
            pltpu.sync_copy(dst_hbm.at[pl.ds(base + j * CHUNK, CHUNK)], dstv)
            pltpu.sync_copy(ones_v, acc.at[dstv], add=True)
            return carry

        lax.fori_loop(0, n_chunks, body, 0)
        plsc.subcore_barrier()
        pltpu.sync_copy(acc.at[pl.ds(s * rows_per_tile, rows_per_tile)],
                        out_hbm.at[c, pl.ds(s * rows_per_tile, rows_per_tile)])

    return deg_kernel


def _make_prop_kernel(n, n_pad, e_pad, d, interpret=False):
    edges_per_tile = e_pad // (NC * NS)
    n_chunks = edges_per_tile // CHUNK
    rows_per_tile = n_pad // NS
    mesh = plsc.VectorSubcoreMesh(core_axis_name="c", subcore_axis_name="s", num_cores=NC, num_subcores=NS)

    @functools.partial(
        pl.kernel,
        mesh=mesh,
        out_type=jax.ShapeDtypeStruct((NC, n_pad, d), jnp.float32),
        scratch_types=[
            pltpu.VMEM_SHARED((n_pad, d), jnp.float32),
            pltpu.VMEM((CHUNK,), jnp.int32),
            pltpu.VMEM((CHUNK,), jnp.int32),
            pltpu.VMEM((CHUNK, d), jnp.float32),
            pltpu.SemaphoreType.DMA,
        ],
        interpret=interpret,
    )
    def prop_kernel(src_hbm, dst_hbm, tbl_hbm, zeros_hbm, out_hbm,
                    acc, srcv, dstv, rows, sem):
        c = lax.axis_index("c")
        s = lax.axis_index("s")
        pltpu.sync_copy(zeros_hbm.at[pl.ds(s * rows_per_tile, rows_per_tile)],
                        acc.at[pl.ds(s * rows_per_tile, rows_per_tile)])
        plsc.subcore_barrier()
        base = c * (e_pad // NC) + s * edges_per_tile

        def body(j, carry):
            off = base + j * CHUNK
            pltpu.sync_copy(src_hbm.at[pl.ds(off, CHUNK)], srcv)
            pltpu.sync_copy(dst_hbm.at[pl.ds(off, CHUNK)], dstv)
            pltpu.async_copy(tbl_hbm.at[srcv], rows, sem).wait()
            pltpu.sync_copy(rows, acc.at[dstv], add=True)
            return carry

        lax.fori_loop(0, n_chunks, body, 0)
        plsc.subcore_barrier()
        pltpu.sync_copy(acc.at[pl.ds(s * rows_per_tile, rows_per_tile)],
                        out_hbm.at[c, pl.ds(s * rows_per_tile, rows_per_tile)])

    return prop_kernel


def _layer_norm(z, g, b):
    mu = jnp.mean(z, axis=-1, keepdims=True)
    var = jnp.var(z, axis=-1, keepdims=True)
    return (z - mu) / jnp.sqrt(var + 1e-5) * g + b


def _prep_body(deg_ref, x_ref, g0_ref, d2_ref, sdeg_ref):
    deg = 1.0 + deg_ref[0, :, 0:1] + deg_ref[1, :, 0:1]
    dinv = lax.rsqrt(deg)
    g0_ref[...] = x_ref[...] * dinv
    d2_ref[...] = 1.0 / deg
    sdeg_ref[...] = jnp.sqrt(deg)


def _combine_body(sp_ref, g_ref, d2_ref, out_ref):
    s = sp_ref[0] + sp_ref[1] + g_ref[...]
    out_ref[...] = s * d2_ref[...]


def _mlp_body(x_ref, g1_ref, g2_ref, g3_ref, sdeg_ref,
              wh_ref, bh_ref, lng_ref, lnb_ref,
              wo_ref, bo_ref, lngo_ref, lnbo_ref, out_ref):
    sdeg = sdeg_ref[...]
    hs = (x_ref[...], g1_ref[...] * sdeg, g2_ref[...] * sdeg,
          g3_ref[...] * sdeg)
    acc = None
    for i in range(4):
        z = jnp.dot(hs[i], wh_ref[i], preferred_element_type=jnp.float32)
        z = z + bh_ref[i:i + 1, :]
        z = _layer_norm(z, lng_ref[i:i + 1, :], lnb_ref[i:i + 1, :])
        z = jnp.maximum(z, 0.0)
        part = jnp.dot(z, wo_ref[i * 128:(i + 1) * 128, :],
                       preferred_element_type=jnp.float32)
        acc = part if acc is None else acc + part
    z = acc + bo_ref[...]
    z = _layer_norm(z, lngo_ref[...], lnbo_ref[...])
    out_ref[...] = jnp.maximum(z, 0.0)


def kernel(x, edge_index, W_hops, b_hops, ln_g_hops, ln_b_hops,
           W_out, b_out, ln_g_out, ln_b_out):
    n, d = x.shape
    e = edge_index.shape[1]
    n_pad = _round_up(n, NS * 8)
    e_pad = _round_up(e, NC * NS * CHUNK)

    src = jnp.concatenate([edge_index[0],
                           jnp.zeros((e_pad - e,), jnp.int32)])
    dst = jnp.concatenate([edge_index[1],
                           jnp.full((e_pad - e,), n, jnp.int32)])

    ones_d = jnp.ones((CHUNK, d), jnp.float32)
    zeros_d = jnp.zeros((n_pad, d), jnp.float32)

    deg_parts = _make_deg_kernel(n_pad, e_pad, d)(dst, ones_d, zeros_d)
    prop = _make_prop_kernel(n, n_pad, e_pad, d)

    grid_r = 400
    grid = (n // grid_r,)
    row_spec = pl.BlockSpec((grid_r, d), lambda i: (i, 0))
    col_spec = pl.BlockSpec((grid_r, 1), lambda i: (i, 0))

    g0, d2, sdeg = pl.pallas_call(
        _prep_body,
        grid=grid,
        in_specs=[pl.BlockSpec((2, grid_r, d), lambda i: (0, i, 0)), row_spec],
        out_specs=[row_spec, col_spec, col_spec],
        out_shape=[jax.ShapeDtypeStruct((n, d), jnp.float32),
                   jax.ShapeDtypeStruct((n, 1), jnp.float32),
                   jax.ShapeDtypeStruct((n, 1), jnp.float32)],
    )(deg_parts[:, :n, :], x)

    combine = pl.pallas_call(
        _combine_body,
        grid=grid,
        in_specs=[pl.BlockSpec((2, grid_r, d), lambda i: (0, i, 0)),
                  row_spec, col_spec],
        out_specs=row_spec,
        out_shape=jax.ShapeDtypeStruct((n, d), jnp.float32),
    )

    gs = [g0]
    for _ in range(3):
        s_parts = prop(src, dst, gs[-1], zeros_d)
        gs.append(combine(s_parts[:, :n, :], gs[-1], d2))

    full = lambda *shape: pl.BlockSpec(shape, lambda i: (0,) * len(shape))
    out = pl.pallas_call(
        _mlp_body,
        grid=grid,
        in_specs=[row_spec, row_spec, row_spec, row_spec, col_spec,
                  full(4, 128, 128), full(4, 128), full(4, 128), full(4, 128),
                  full(512, 128), full(1, 128), full(1, 128), full(1, 128)],
        out_specs=row_spec,
        out_shape=jax.ShapeDtypeStruct((n, d), jnp.float32),
    )(x, gs[1], gs[2], gs[3], sdeg,
      W_hops, b_hops, ln_g_hops, ln_b_hops,
      W_out, b_out.reshape(1, -1), ln_g_out.reshape(1, -1),
      ln_b_out.reshape(1, -1))
    return out
